# baseline (speedup 1.0000x reference)
import functools

import jax
import jax.numpy as jnp
from jax import lax
from jax.experimental import pallas as pl
from jax.experimental.pallas import tpu as pltpu
from jax.experimental.pallas import tpu_sc as plsc


def kernel(input_tensor, table):
    B, H = input_tensor.shape
    V, D = table.shape
    Hp = (H + 7) // 8 * 8

    info = plsc.get_sparse_core_info()
    NC, NS = info.num_cores, info.num_subcores
    NW = NC * NS

    assert D % NW == 0
    d_per_w = D // NW
    L = 16

    idxT = jnp.pad(input_tensor.T.astype(jnp.int32), ((0, Hp - H), (0, 0)))
    tableT = table.T

    mesh = plsc.VectorSubcoreMesh(core_axis_name="c", subcore_axis_name="s")

    @functools.partial(
        pl.kernel,
        out_type=jax.ShapeDtypeStruct((H, D, B), jnp.float32),
        mesh=mesh,
        scratch_types=[
            pltpu.VMEM((V,), jnp.float32),
            pltpu.VMEM((B,), jnp.int32),
            pltpu.VMEM((2, B), jnp.float32),
            pltpu.SemaphoreType.DMA,
            pltpu.SemaphoreType.DMA,
            [pltpu.SemaphoreType.DMA] * 2,
        ],
        compiler_params=pltpu.CompilerParams(
            use_tc_tiling_on_sc=False, needs_layout_passes=False
        ),
    )
    def emb(idxT_hbm, tableT_hbm, out_hbm, trow_v, irow_v, gath_v, tsem, isem, osems):
        wid = lax.axis_index("s") * NC + lax.axis_index("c")

        def ostart(h, d, j):
            pltpu.async_copy(gath_v.at[j], out_hbm.at[h, d], osems[j])

        def owait(h, d, j):
            pltpu.make_async_copy(gath_v.at[j], out_hbm.at[h, d], osems[j]).wait()

        for dd in range(d_per_w):
            d = wid * d_per_w + dd
            pltpu.async_copy(tableT_hbm.at[d], trow_v, tsem).wait()

            def hloop(hg, carry):
                for j in range(2):
                    h = hg * 2 + j
                    pltpu.async_copy(idxT_hbm.at[h], irow_v, isem).wait()

                    @pl.when(h >= 2)
                    def _():
                        owait(h - 2, d, j)

                    for q in range(B // L):
                        iv = irow_v[pl.ds(L * q, L)]
                        gath_v[j, pl.ds(L * q, L)] = plsc.load_gather(
                            trow_v, [iv]
                        )

                    ostart(h, d, j)
                return carry

            lax.fori_loop(0, H // 2, hloop, 0)
            owait(H - 2, d, 0)
            owait(H - 1, d, 1)

    out_t = emb(idxT, tableT)
    return jnp.transpose(out_t, (2, 0, 1))


# speedup vs baseline: 2.1411x; 2.1411x over previous
import functools

import jax
import jax.numpy as jnp
from jax import lax
from jax.experimental import pallas as pl
from jax.experimental.pallas import tpu as pltpu
from jax.experimental.pallas import tpu_sc as plsc


def kernel(input_tensor, table):
    B, H = input_tensor.shape
    V, D = table.shape
    N = B * H
    Hp = (H + 7) // 8 * 8
    Dp = 128

    info = plsc.get_sparse_core_info()
    NC, NS = info.num_cores, info.num_subcores
    NW = NC * NS

    K = 2 * H
    assert N % (NW * K) == 0
    n_per_w = N // NW
    n_ck = n_per_w // K
    b_per_w = B // NW

    idx = input_tensor.reshape(N // K, K).astype(jnp.int32)

    mesh = plsc.VectorSubcoreMesh(core_axis_name="c", subcore_axis_name="s")

    NBUF = 4
    DEPTH = 2

    @functools.partial(
        pl.kernel,
        out_type=jax.ShapeDtypeStruct((B, Hp, Dp), jnp.float32),
        mesh=mesh,
        scratch_types=[
            pltpu.VMEM((n_ck, K), jnp.int32),
            pltpu.VMEM((NBUF, K, D), jnp.float32),
            pltpu.SemaphoreType.DMA,
            [pltpu.SemaphoreType.DMA] * NBUF,
            [pltpu.SemaphoreType.DMA] * NBUF,
        ],
        compiler_params=pltpu.CompilerParams(use_tc_tiling_on_sc=False),
    )
    def emb(idx_hbm, table_hbm, out_hbm, idx_v, rows_v, isem, gsems, wsems):
        wid = lax.axis_index("s") * NC + lax.axis_index("c")
        b0 = wid * b_per_w
        pltpu.async_copy(idx_hbm.at[pl.ds(wid * n_ck, n_ck)], idx_v, isem).wait()

        def gstart(c, j):
            pltpu.async_copy(
                table_hbm.at[idx_v.at[c]], rows_v.at[j], gsems[j]
            )

        def gwait(c, j):
            pltpu.make_async_copy(
                table_hbm.at[idx_v.at[c]], rows_v.at[j], gsems[j]
            ).wait()

        def wstart(c, j):
            b = b0 + 2 * c
            pltpu.async_copy(
                rows_v.at[j, pl.ds(0, H)],
                out_hbm.at[b, pl.ds(0, H), pl.ds(0, D)],
                wsems[j],
            )
            pltpu.async_copy(
                rows_v.at[j, pl.ds(H, H)],
                out_hbm.at[b + 1, pl.ds(0, H), pl.ds(0, D)],
                wsems[j],
            )

        def wwait(c, j):
            b = b0 + 2 * c
            pltpu.make_async_copy(
                rows_v.at[j, pl.ds(0, H)],
                out_hbm.at[b, pl.ds(0, H), pl.ds(0, D)],
                wsems[j],
            ).wait()
            pltpu.make_async_copy(
                rows_v.at[j, pl.ds(H, H)],
                out_hbm.at[b + 1, pl.ds(0, H), pl.ds(0, D)],
                wsems[j],
            ).wait()

        for d in range(DEPTH):
            gstart(d, d)

        def body(gi, carry):
            base = gi * NBUF
            for j in range(NBUF):
                c = base + j
                jj = (j + DEPTH) % NBUF

                @pl.when(c >= NBUF - DEPTH)
                def _():
                    wwait(c - (NBUF - DEPTH), jj)

                @pl.when(c + DEPTH < n_ck)
                def _():
                    gstart(c + DEPTH, jj)

                gwait(c, j)
                wstart(c, j)
            return carry

        lax.fori_loop(0, n_ck // NBUF, body, 0)
        for c in range(n_ck - (NBUF - DEPTH), n_ck):
            wwait(c, c % NBUF)

    out_p = emb(idx, table)
    return out_p[:, :H, :D]


# speedup vs baseline: 2.1459x; 1.0023x over previous
import functools

import jax
import jax.numpy as jnp
from jax import lax
from jax.experimental import pallas as pl
from jax.experimental.pallas import tpu as pltpu
from jax.experimental.pallas import tpu_sc as plsc


def kernel(input_tensor, table):
    B, H = input_tensor.shape
    V, D = table.shape
    N = B * H
    Hp = (H + 7) // 8 * 8
    Dp = 128

    info = plsc.get_sparse_core_info()
    NC, NS = info.num_cores, info.num_subcores
    NW = NC * NS

    K = 2 * H
    assert N % (NW * K) == 0
    n_per_w = N // NW
    n_ck = n_per_w // K
    b_per_w = B // NW

    idx = input_tensor.reshape(N // K, K).astype(jnp.int32)

    mesh = plsc.VectorSubcoreMesh(core_axis_name="c", subcore_axis_name="s")

    NBUF = 8
    DEPTH = 3

    @functools.partial(
        pl.kernel,
        out_type=jax.ShapeDtypeStruct((B, Hp, Dp), jnp.float32),
        mesh=mesh,
        scratch_types=[
            pltpu.VMEM((n_ck, K), jnp.int32),
            pltpu.VMEM((NBUF, K, D), jnp.float32),
            pltpu.SemaphoreType.DMA,
            [pltpu.SemaphoreType.DMA] * NBUF,
            [pltpu.SemaphoreType.DMA] * NBUF,
        ],
        compiler_params=pltpu.CompilerParams(use_tc_tiling_on_sc=False),
    )
    def emb(idx_hbm, table_hbm, out_hbm, idx_v, rows_v, isem, gsems, wsems):
        wid = lax.axis_index("s") * NC + lax.axis_index("c")
        b0 = wid * b_per_w
        pltpu.async_copy(idx_hbm.at[pl.ds(wid * n_ck, n_ck)], idx_v, isem).wait()

        def gstart(c, j):
            pltpu.async_copy(
                table_hbm.at[idx_v.at[c]], rows_v.at[j], gsems[j]
            )

        def gwait(c, j):
            pltpu.make_async_copy(
                table_hbm.at[idx_v.at[c]], rows_v.at[j], gsems[j]
            ).wait()

        def wstart(c, j):
            b = b0 + 2 * c
            pltpu.async_copy(
                rows_v.at[j, pl.ds(0, H)],
                out_hbm.at[b, pl.ds(0, H), pl.ds(0, D)],
                wsems[j],
            )
            pltpu.async_copy(
                rows_v.at[j, pl.ds(H, H)],
                out_hbm.at[b + 1, pl.ds(0, H), pl.ds(0, D)],
                wsems[j],
            )

        def wwait(c, j):
            b = b0 + 2 * c
            pltpu.make_async_copy(
                rows_v.at[j, pl.ds(0, H)],
                out_hbm.at[b, pl.ds(0, H), pl.ds(0, D)],
                wsems[j],
            ).wait()
            pltpu.make_async_copy(
                rows_v.at[j, pl.ds(H, H)],
                out_hbm.at[b + 1, pl.ds(0, H), pl.ds(0, D)],
                wsems[j],
            ).wait()

        for d in range(DEPTH):
            gstart(d, d)

        def body(gi, carry):
            base = gi * NBUF
            for j in range(NBUF):
                c = base + j
                jj = (j + DEPTH) % NBUF

                @pl.when(c >= NBUF - DEPTH)
                def _():
                    wwait(c - (NBUF - DEPTH), jj)

                @pl.when(c + DEPTH < n_ck)
                def _():
                    gstart(c + DEPTH, jj)

                gwait(c, j)
                wstart(c, j)
            return carry

        lax.fori_loop(0, n_ck // NBUF, body, 0)
        for c in range(n_ck - (NBUF - DEPTH), n_ck):
            wwait(c, c % NBUF)

    out_p = emb(idx, table)
    return out_p[:, :H, :D]


# speedup vs baseline: 2.1591x; 1.0062x over previous
import functools

import jax
import jax.numpy as jnp
from jax import lax
from jax.experimental import pallas as pl
from jax.experimental.pallas import tpu as pltpu
from jax.experimental.pallas import tpu_sc as plsc


def kernel(input_tensor, table):
    B, H = input_tensor.shape
    V, D = table.shape
    N = B * H
    Hp = (H + 7) // 8 * 8
    Dp = 128

    info = plsc.get_sparse_core_info()
    NC, NS = info.num_cores, info.num_subcores
    NW = NC * NS

    K = 2 * H
    assert N % (NW * K) == 0
    n_per_w = N // NW
    n_ck = n_per_w // K
    b_per_w = B // NW

    idx = input_tensor.reshape(N // K, K).astype(jnp.int32)

    mesh = plsc.VectorSubcoreMesh(core_axis_name="c", subcore_axis_name="s")

    NBUF = 8
    DEPTH = 5

    @functools.partial(
        pl.kernel,
        out_type=jax.ShapeDtypeStruct((B, Hp, Dp), jnp.float32),
        mesh=mesh,
        scratch_types=[
            pltpu.VMEM((n_ck, K), jnp.int32),
            pltpu.VMEM((NBUF, K, D), jnp.float32),
            pltpu.SemaphoreType.DMA,
            [pltpu.SemaphoreType.DMA] * NBUF,
            [pltpu.SemaphoreType.DMA] * NBUF,
        ],
        compiler_params=pltpu.CompilerParams(use_tc_tiling_on_sc=False),
    )
    def emb(idx_hbm, table_hbm, out_hbm, idx_v, rows_v, isem, gsems, wsems):
        wid = lax.axis_index("s") * NC + lax.axis_index("c")
        b0 = wid * b_per_w
        pltpu.async_copy(idx_hbm.at[pl.ds(wid * n_ck, n_ck)], idx_v, isem).wait()

        def gstart(c, j):
            pltpu.async_copy(
                table_hbm.at[idx_v.at[c]], rows_v.at[j], gsems[j]
            )

        def gwait(c, j):
            pltpu.make_async_copy(
                table_hbm.at[idx_v.at[c]], rows_v.at[j], gsems[j]
            ).wait()

        def wstart(c, j):
            b = b0 + 2 * c
            pltpu.async_copy(
                rows_v.at[j, pl.ds(0, H)],
                out_hbm.at[b, pl.ds(0, H), pl.ds(0, D)],
                wsems[j],
            )
            pltpu.async_copy(
                rows_v.at[j, pl.ds(H, H)],
                out_hbm.at[b + 1, pl.ds(0, H), pl.ds(0, D)],
                wsems[j],
            )

        def wwait(c, j):
            b = b0 + 2 * c
            pltpu.make_async_copy(
                rows_v.at[j, pl.ds(0, H)],
                out_hbm.at[b, pl.ds(0, H), pl.ds(0, D)],
                wsems[j],
            ).wait()
            pltpu.make_async_copy(
                rows_v.at[j, pl.ds(H, H)],
                out_hbm.at[b + 1, pl.ds(0, H), pl.ds(0, D)],
                wsems[j],
            ).wait()

        for d in range(DEPTH):
            gstart(d, d)

        def body(gi, carry):
            base = gi * NBUF
            for j in range(NBUF):
                c = base + j
                jj = (j + DEPTH) % NBUF

                @pl.when(c >= NBUF - DEPTH)
                def _():
                    wwait(c - (NBUF - DEPTH), jj)

                @pl.when(c + DEPTH < n_ck)
                def _():
                    gstart(c + DEPTH, jj)

                gwait(c, j)
                wstart(c, j)
            return carry

        lax.fori_loop(0, n_ck // NBUF, body, 0)
        for c in range(n_ck - (NBUF - DEPTH), n_ck):
            wwait(c, c % NBUF)

    out_p = emb(idx, table)
    return out_p[:, :H, :D]
